# Initial kernel scaffold; baseline (speedup 1.0000x reference)
import jax
import jax.numpy as jnp
from jax.experimental import pallas as pl


def kernel(x, conv0_w4, conv0_b4, conv1_w4, conv1_b4, tail_w1, tail_b1, tail_w2, tail_b2, tail_wl, tail_bl):
    raise NotImplementedError("write your pallas kernel here")


# speedup vs baseline: 1.2547x; 1.2547x over previous
import jax
import jax.numpy as jnp
from jax.experimental import pallas as pl
from jax.experimental.pallas import tpu as pltpu

_SLOPE = 0.01


def _lrelu(v):
    return jnp.where(v >= 0, v, _SLOPE * v)


def _fused_body(f0_ref, f1_ref, w0_ref, b0_ref, w1_ref, b1_ref,
                tw1_ref, tb1_ref, tw2_ref, tb2_ref, twl_ref, tbl_ref,
                o_ref, cat0, acc0, h0, sp1, acc1, hp, col):
    B = o_ref.shape[0]
    Cp = w0_ref.shape[2]
    M0 = B * 81
    M1 = B * 25
    ext = f1_ref.shape[0]

    acc0[...] = jnp.dot(f0_ref[...], w0_ref[0],
                        preferred_element_type=jnp.float32)
    cat0[0:M0, :] = f0_ref[...]
    cat0[M0:M0 + ext, :] = f1_ref[...]
    for t, off in ((1, 1), (2, 9), (3, 10)):
        acc0[...] += jnp.dot(cat0[off:off + M0, :], w0_ref[t],
                             preferred_element_type=jnp.float32)
    h0[...] = _lrelu(acc0[...] + b0_ref[...]).astype(jnp.bfloat16)

    sp1[...] = jnp.zeros(sp1.shape, sp1.dtype)
    hv = h0[...].reshape(B, 9, 9, Cp)[:, :8, :8, :].reshape(B, 4, 2, 4, 2, Cp)
    for p in range(2):
        for q in range(2):
            sp1[0:B, 1:5, 1:5, (2 * p + q) * Cp:(2 * p + q + 1) * Cp] = \
                hv[:, :, p, :, q, :]

    v1 = sp1[...].reshape((B + 1) * 25, 4 * Cp)
    acc1[...] = jnp.dot(v1[0:M1], w1_ref[0], preferred_element_type=jnp.float32)
    for t, off in ((1, 1), (2, 5), (3, 6)):
        acc1[...] += jnp.dot(v1[off:off + M1], w1_ref[t],
                             preferred_element_type=jnp.float32)
    h1 = _lrelu(acc1[...] + b1_ref[...]).astype(jnp.bfloat16).reshape(B, 5, 5, Cp)

    hp[...] = jnp.zeros(hp.shape, hp.dtype)
    hp[0:B, 1:5, 1:5, :] = h1[:, 0:4, 0:4, :]
    acc2 = None
    for i in (0, 1):
        for j in (0, 1):
            a = 2 * i + j
            col[...] = hp[0:B, 2 * i:2 * i + 3, 2 * j:2 * j + 3, :] \
                .reshape(B, 9 * Cp)
            h = jnp.dot(col[...], tw1_ref[...],
                        preferred_element_type=jnp.float32) + tb1_ref[...]
            h = _lrelu(h).astype(jnp.bfloat16)
            c = jnp.dot(h, tw2_ref[a], preferred_element_type=jnp.float32)
            acc2 = c if acc2 is None else acc2 + c
    h2 = _lrelu(acc2 + tb2_ref[...]).astype(jnp.bfloat16)
    o_ref[...] = jnp.dot(h2, twl_ref[...],
                         preferred_element_type=jnp.float32) + tbl_ref[...]


def kernel(x, conv0_w4, conv0_b4, conv1_w4, conv1_b4,
           tail_w1, tail_b1, tail_w2, tail_b2, tail_wl, tail_bl):
    N, Cin, H, W = x.shape
    assert H == 16 and W == 16 and N % 2 == 0, (N, Cin, H, W)
    K0 = conv0_w4.shape[1]
    Cp = conv0_w4.shape[2]
    assert K0 == 4 * Cin and conv1_w4.shape == (4, 4 * Cp, Cp)
    assert tail_w1.shape == (9 * Cp, Cp) and tail_wl.shape == (Cp, Cp)
    B = N // 2
    M0 = B * 81
    ext = 16
    assert M0 % ext == 0

    xh = jnp.transpose(x, (0, 2, 3, 1))
    s = xh.reshape(N, 8, 2, 8, 2, Cin).transpose(0, 1, 3, 2, 4, 5)
    s = s.reshape(N, 8, 8, K0)
    sp = jnp.pad(s, ((0, 0), (1, 0), (1, 0), (0, 0)))
    F = sp.reshape(N * 81, K0)
    F = jnp.pad(F, ((0, ext), (0, 0))).astype(jnp.bfloat16)

    wspec = pl.BlockSpec(memory_space=pltpu.MemorySpace.VMEM)
    out = pl.pallas_call(
        _fused_body,
        out_shape=jax.ShapeDtypeStruct((N, Cp), jnp.float32),
        grid=(2,),
        in_specs=[
            pl.BlockSpec((M0, K0), lambda i: (i, 0)),
            pl.BlockSpec((ext, K0), lambda i: ((M0 // ext) * (i + 1), 0)),
            wspec, wspec, wspec, wspec, wspec, wspec, wspec, wspec, wspec, wspec,
        ],
        out_specs=pl.BlockSpec((B, Cp), lambda i: (i, 0)),
        scratch_shapes=[
            pltpu.VMEM((M0 + ext, K0), jnp.bfloat16),
            pltpu.VMEM((M0, Cp), jnp.float32),
            pltpu.VMEM((M0, Cp), jnp.bfloat16),
            pltpu.VMEM((B + 1, 5, 5, 4 * Cp), jnp.bfloat16),
            pltpu.VMEM((B * 25, Cp), jnp.float32),
            pltpu.VMEM((B, 6, 6, Cp), jnp.bfloat16),
            pltpu.VMEM((B, 9 * Cp), jnp.bfloat16),
        ],
        compiler_params=pltpu.CompilerParams(
            dimension_semantics=("parallel",),
            vmem_limit_bytes=int(62 << 20)),
        cost_estimate=pl.CostEstimate(
            flops=int(8 * (M0 * K0 + B * 25 * 4 * Cp + B * 9 * Cp) * Cp * 2),
            transcendentals=0,
            bytes_accessed=int(2 * (N * 81 * K0 + 4 * K0 * Cp + 20 * Cp * Cp))),
    )(F, F, conv0_w4, conv0_b4, conv1_w4, conv1_b4,
      tail_w1, tail_b1, tail_w2, tail_b2, tail_wl, tail_bl)
    return out


# speedup vs baseline: 1.9855x; 1.5824x over previous
import jax
import jax.numpy as jnp
from jax.experimental import pallas as pl
from jax.experimental.pallas import tpu as pltpu

_SLOPE = 0.01


def _lrelu(v):
    return jnp.where(v >= 0, v, _SLOPE * v)


def _fused_body(xb_ref, w0_ref, b0_ref, w1_hbm, b1_ref,
                tw1_hbm, tb1_ref, tw2_hbm, tb2_ref, twl_hbm, tbl_ref,
                o_ref,
                w1s, tw1s, tw2s, twls, sems,
                xt, cat0, acc0, h0, sp1, acc1, hp, col):
    B = o_ref.shape[0]
    Cin = xb_ref.shape[1]
    K0 = 4 * Cin
    Cp = w0_ref.shape[2]
    M0 = B * 81
    M1 = B * 25

    cp1 = pltpu.make_async_copy(w1_hbm, w1s, sems.at[0])
    cp2 = pltpu.make_async_copy(tw1_hbm, tw1s, sems.at[1])
    cp3 = pltpu.make_async_copy(tw2_hbm, tw2s, sems.at[2])
    cp4 = pltpu.make_async_copy(twl_hbm, twls, sems.at[3])
    cp1.start()
    cp2.start()
    cp3.start()
    cp4.start()

    ii = jax.lax.broadcasted_iota(jnp.int32, (Cin, Cin), 0)
    jj = jax.lax.broadcasted_iota(jnp.int32, (Cin, Cin), 1)
    ident = (ii == jj).astype(jnp.bfloat16)
    for n in range(B):
        xt[n] = jax.lax.dot_general(
            xb_ref[n], ident, (((0,), (0,)), ((), ())),
            preferred_element_type=jnp.float32).astype(jnp.bfloat16)

    half = B // 2
    for hfi in range(2):
        xv = xt[hfi * half:(hfi + 1) * half].reshape(half, 8, 2, 8, 2, Cin)
        sall = jnp.concatenate(
            [xv[:, :, p, :, q, :] for p in (0, 1) for q in (0, 1)], axis=-1)
        sall = jnp.pad(sall, ((0, 0), (1, 0), (1, 0), (0, 0)))
        cat0[hfi * half * 81:(hfi + 1) * half * 81, :] = \
            sall.reshape(half * 81, K0)
    cat0[M0:M0 + 16, :] = jnp.zeros((16, K0), jnp.bfloat16)

    acc0[...] = jnp.dot(cat0[0:M0, :], w0_ref[0],
                        preferred_element_type=jnp.float32)
    for t, off in ((1, 1), (2, 9), (3, 10)):
        acc0[...] += jnp.dot(cat0[off:off + M0, :], w0_ref[t],
                             preferred_element_type=jnp.float32)
    h0[...] = _lrelu(acc0[...] + b0_ref[...]).astype(jnp.bfloat16)

    sp1[...] = jnp.zeros(sp1.shape, sp1.dtype)
    hv = h0[...].reshape(B, 9, 9, Cp)[:, :8, :8, :].reshape(B, 4, 2, 4, 2, Cp)
    for p in range(2):
        for q in range(2):
            sp1[0:B, 1:5, 1:5, (2 * p + q) * Cp:(2 * p + q + 1) * Cp] = \
                hv[:, :, p, :, q, :]

    cp1.wait()
    v1 = sp1[...].reshape((B + 1) * 25, 4 * Cp)
    acc1[...] = jnp.dot(v1[0:M1], w1s[0], preferred_element_type=jnp.float32)
    for t, off in ((1, 1), (2, 5), (3, 6)):
        acc1[...] += jnp.dot(v1[off:off + M1], w1s[t],
                             preferred_element_type=jnp.float32)
    h1 = _lrelu(acc1[...] + b1_ref[...]).astype(jnp.bfloat16).reshape(B, 5, 5, Cp)

    hp[...] = jnp.zeros(hp.shape, hp.dtype)
    hp[0:B, 1:5, 1:5, :] = h1[:, 0:4, 0:4, :]
    cp2.wait()
    cp3.wait()
    cp4.wait()
    acc2 = None
    for i in (0, 1):
        for j in (0, 1):
            a = 2 * i + j
            col[...] = hp[0:B, 2 * i:2 * i + 3, 2 * j:2 * j + 3, :] \
                .reshape(B, 9 * Cp)
            h = jnp.dot(col[...], tw1s[...],
                        preferred_element_type=jnp.float32) + tb1_ref[...]
            h = _lrelu(h).astype(jnp.bfloat16)
            c = jnp.dot(h, tw2s[a], preferred_element_type=jnp.float32)
            acc2 = c if acc2 is None else acc2 + c
    h2 = _lrelu(acc2 + tb2_ref[...]).astype(jnp.bfloat16)
    o_ref[...] = jnp.dot(h2, twls[...],
                         preferred_element_type=jnp.float32) + tbl_ref[...]


def kernel(x, conv0_w4, conv0_b4, conv1_w4, conv1_b4,
           tail_w1, tail_b1, tail_w2, tail_b2, tail_wl, tail_bl):
    N, Cin, H, W = x.shape
    assert H == 16 and W == 16 and N % 2 == 0, (N, Cin, H, W)
    K0 = conv0_w4.shape[1]
    Cp = conv0_w4.shape[2]
    assert K0 == 4 * Cin and conv1_w4.shape == (4, 4 * Cp, Cp)
    assert tail_w1.shape == (9 * Cp, Cp) and tail_wl.shape == (Cp, Cp)
    B = N // 2
    assert B % 2 == 0
    M0 = B * 81

    xb = x.reshape(N, Cin, H * W).astype(jnp.bfloat16)

    vspec = pl.BlockSpec(memory_space=pltpu.MemorySpace.VMEM)
    aspec = pl.BlockSpec(memory_space=pltpu.MemorySpace.HBM)
    out = pl.pallas_call(
        _fused_body,
        out_shape=jax.ShapeDtypeStruct((N, Cp), jnp.float32),
        grid=(2,),
        in_specs=[
            pl.BlockSpec((B, Cin, H * W), lambda i: (i, 0, 0)),
            vspec, vspec, aspec, vspec,
            aspec, vspec, aspec, vspec, aspec, vspec,
        ],
        out_specs=pl.BlockSpec((B, Cp), lambda i: (i, 0)),
        scratch_shapes=[
            pltpu.VMEM((4, 4 * Cp, Cp), jnp.bfloat16),
            pltpu.VMEM((9 * Cp, Cp), jnp.bfloat16),
            pltpu.VMEM((4, Cp, Cp), jnp.bfloat16),
            pltpu.VMEM((Cp, Cp), jnp.bfloat16),
            pltpu.SemaphoreType.DMA((4,)),
            pltpu.VMEM((N // 2, H * W, Cin), jnp.bfloat16),
            pltpu.VMEM((M0 + 16, K0), jnp.bfloat16),
            pltpu.VMEM((M0, Cp), jnp.float32),
            pltpu.VMEM((M0, Cp), jnp.bfloat16),
            pltpu.VMEM((N // 2 + 1, 5, 5, 4 * Cp), jnp.bfloat16),
            pltpu.VMEM((N // 2 * 25, Cp), jnp.float32),
            pltpu.VMEM((N // 2, 6, 6, Cp), jnp.bfloat16),
            pltpu.VMEM((N // 2, 9 * Cp), jnp.bfloat16),
        ],
        compiler_params=pltpu.CompilerParams(
            dimension_semantics=("parallel",),
            vmem_limit_bytes=int(62 << 20)),
        cost_estimate=pl.CostEstimate(
            flops=int(8 * (M0 * K0 + (N // 2) * 25 * 4 * Cp
                           + (N // 2) * 9 * Cp) * Cp * 2
                      + 4 * N * H * W * Cin * Cin),
            transcendentals=0,
            bytes_accessed=int(N * Cin * H * W * 2 + 2 * 24 * Cp * Cp)),
    )(xb, conv0_w4, conv0_b4, conv1_w4, conv1_b4,
      tail_w1, tail_b1, tail_w2, tail_b2, tail_wl, tail_bl)
    return out


# speedup vs baseline: 2.4490x; 1.2334x over previous
import jax
import jax.numpy as jnp
from jax.experimental import pallas as pl
from jax.experimental.pallas import tpu as pltpu

_SLOPE = 0.01


def _lrelu(v):
    return jnp.where(v >= 0, v, _SLOPE * v)


def _fused_body(xb_ref, w0_ref, b0_ref, w1_hbm, b1_ref,
                tw1_hbm, tb1_ref, tw2_hbm, tb2_ref, twl_hbm, tbl_ref,
                o_ref,
                w1s, tw1s, tw2s, twls, sems,
                cat0, acc0, h0, sp1, acc1, hp, col):
    B = o_ref.shape[0]
    Cin = xb_ref.shape[1]
    K0 = 4 * Cin
    Cp = w0_ref.shape[2]
    M0 = B * 81
    M1 = B * 25

    cp1 = pltpu.make_async_copy(w1_hbm, w1s, sems.at[0])
    cp2 = pltpu.make_async_copy(tw1_hbm, tw1s, sems.at[1])
    cp3 = pltpu.make_async_copy(tw2_hbm, tw2s, sems.at[2])
    cp4 = pltpu.make_async_copy(twl_hbm, twls, sems.at[3])
    cp1.start()
    cp2.start()
    cp3.start()
    cp4.start()

    ri = jax.lax.broadcasted_iota(jnp.int32, (256, 256), 0)
    ci = jax.lax.broadcasted_iota(jnp.int32, (256, 256), 1)
    p_, q_ = (ri >> 7) & 1, (ri >> 6) & 1
    a_, b_ = (ri >> 3) & 7, ri & 7
    perm = ((2 * a_ + p_) * 16 + 2 * b_ + q_ == ci).astype(xb_ref.dtype)

    cat0[...] = jnp.zeros(cat0.shape, cat0.dtype)
    for n in range(B):
        y = jax.lax.dot_general(perm, xb_ref[n], (((1,), (1,)), ((), ())),
                                preferred_element_type=jnp.float32)
        y = y.astype(jnp.bfloat16)
        for a in range(8):
            cat0[n * 81 + 9 * (a + 1) + 1:n * 81 + 9 * (a + 1) + 9, :] = \
                jnp.concatenate([y[t * 64 + a * 8:t * 64 + a * 8 + 8, :]
                                 for t in range(4)], axis=1)

    acc0[...] = jnp.dot(cat0[0:M0, :], w0_ref[0],
                        preferred_element_type=jnp.float32)
    for t, off in ((1, 1), (2, 9), (3, 10)):
        acc0[...] += jnp.dot(cat0[off:off + M0, :], w0_ref[t],
                             preferred_element_type=jnp.float32)
    h0[...] = _lrelu(acc0[...] + b0_ref[...]).astype(jnp.bfloat16)

    sp1[...] = jnp.zeros(sp1.shape, sp1.dtype)
    hv = h0[...].reshape(B, 9, 9, Cp)[:, :8, :8, :].reshape(B, 4, 2, 4, 2, Cp)
    for p in range(2):
        for q in range(2):
            sp1[0:B, 1:5, 1:5, (2 * p + q) * Cp:(2 * p + q + 1) * Cp] = \
                hv[:, :, p, :, q, :]

    cp1.wait()
    v1 = sp1[...].reshape((B + 1) * 25, 4 * Cp)
    acc1[...] = jnp.dot(v1[0:M1], w1s[0], preferred_element_type=jnp.float32)
    for t, off in ((1, 1), (2, 5), (3, 6)):
        acc1[...] += jnp.dot(v1[off:off + M1], w1s[t],
                             preferred_element_type=jnp.float32)
    h1 = _lrelu(acc1[...] + b1_ref[...]).astype(jnp.bfloat16).reshape(B, 5, 5, Cp)

    hp[...] = jnp.zeros(hp.shape, hp.dtype)
    hp[0:B, 1:5, 1:5, :] = h1[:, 0:4, 0:4, :]
    cp2.wait()
    cp3.wait()
    cp4.wait()
    acc2 = None
    for i in (0, 1):
        for j in (0, 1):
            a = 2 * i + j
            col[...] = hp[0:B, 2 * i:2 * i + 3, 2 * j:2 * j + 3, :] \
                .reshape(B, 9 * Cp)
            h = jnp.dot(col[...], tw1s[...],
                        preferred_element_type=jnp.float32) + tb1_ref[...]
            h = _lrelu(h).astype(jnp.bfloat16)
            c = jnp.dot(h, tw2s[a], preferred_element_type=jnp.float32)
            acc2 = c if acc2 is None else acc2 + c
    h2 = _lrelu(acc2 + tb2_ref[...]).astype(jnp.bfloat16)
    o_ref[...] = jnp.dot(h2, twls[...],
                         preferred_element_type=jnp.float32) + tbl_ref[...]


def kernel(x, conv0_w4, conv0_b4, conv1_w4, conv1_b4,
           tail_w1, tail_b1, tail_w2, tail_b2, tail_wl, tail_bl):
    N, Cin, H, W = x.shape
    assert H == 16 and W == 16 and N % 4 == 0, (N, Cin, H, W)
    K0 = conv0_w4.shape[1]
    Cp = conv0_w4.shape[2]
    assert K0 == 4 * Cin and conv1_w4.shape == (4, 4 * Cp, Cp)
    assert tail_w1.shape == (9 * Cp, Cp) and tail_wl.shape == (Cp, Cp)
    G = 4
    B = N // G
    M0 = B * 81

    xb = x.reshape(N, Cin, H * W)

    vspec = pl.BlockSpec(memory_space=pltpu.MemorySpace.VMEM)
    aspec = pl.BlockSpec(memory_space=pltpu.MemorySpace.HBM)
    out = pl.pallas_call(
        _fused_body,
        out_shape=jax.ShapeDtypeStruct((N, Cp), jnp.float32),
        grid=(G,),
        in_specs=[
            pl.BlockSpec((B, Cin, H * W), lambda i: (i, 0, 0)),
            vspec, vspec, aspec, vspec,
            aspec, vspec, aspec, vspec, aspec, vspec,
        ],
        out_specs=pl.BlockSpec((B, Cp), lambda i: (i, 0)),
        scratch_shapes=[
            pltpu.VMEM((4, 4 * Cp, Cp), jnp.bfloat16),
            pltpu.VMEM((9 * Cp, Cp), jnp.bfloat16),
            pltpu.VMEM((4, Cp, Cp), jnp.bfloat16),
            pltpu.VMEM((Cp, Cp), jnp.bfloat16),
            pltpu.SemaphoreType.DMA((4,)),
            pltpu.VMEM((M0 + 16, K0), jnp.bfloat16),
            pltpu.VMEM((M0, Cp), jnp.float32),
            pltpu.VMEM((M0, Cp), jnp.bfloat16),
            pltpu.VMEM((B + 1, 5, 5, 4 * Cp), jnp.bfloat16),
            pltpu.VMEM((B * 25, Cp), jnp.float32),
            pltpu.VMEM((B, 6, 6, Cp), jnp.bfloat16),
            pltpu.VMEM((B, 9 * Cp), jnp.bfloat16),
        ],
        compiler_params=pltpu.CompilerParams(
            dimension_semantics=("parallel",),
            vmem_limit_bytes=int(62 << 20)),
        cost_estimate=pl.CostEstimate(
            flops=int(8 * (M0 * K0 + B * 25 * 4 * Cp + B * 9 * Cp) * Cp * 2 * G
                      // G + 2 * N * H * W * H * W * Cin),
            transcendentals=0,
            bytes_accessed=int(N * Cin * H * W * 4 + 2 * 24 * Cp * Cp)),
    )(xb, conv0_w4, conv0_b4, conv1_w4, conv1_b4,
      tail_w1, tail_b1, tail_w2, tail_b2, tail_wl, tail_bl)
    return out


# speedup vs baseline: 2.7092x; 1.1063x over previous
import jax
import jax.numpy as jnp
from jax.experimental import pallas as pl
from jax.experimental.pallas import tpu as pltpu

_SLOPE = 0.01


def _lrelu(v):
    return jnp.where(v >= 0, v, _SLOPE * v)


def _fused_body(xb_ref, w0_ref, b0_ref, w1_hbm, b1_ref,
                tw1_hbm, tb1_ref, tw2_hbm, tb2_ref, twl_hbm, tbl_ref,
                o_ref,
                w1s, tw1s, tw2s, twls, sems,
                cat0, acc0, h0, sp1, acc1, hp, col):
    B = o_ref.shape[0]
    Cin = xb_ref.shape[1]
    K0 = 4 * Cin
    Cp = w0_ref.shape[2]
    M0 = B * 81
    M1 = B * 25

    cp1 = pltpu.make_async_copy(w1_hbm, w1s, sems.at[0])
    cp2 = pltpu.make_async_copy(tw1_hbm, tw1s, sems.at[1])
    cp3 = pltpu.make_async_copy(tw2_hbm, tw2s, sems.at[2])
    cp4 = pltpu.make_async_copy(twl_hbm, twls, sems.at[3])
    cp1.start()
    cp2.start()
    cp3.start()
    cp4.start()

    ri = jax.lax.broadcasted_iota(jnp.int32, (256, 256), 0)
    ci = jax.lax.broadcasted_iota(jnp.int32, (256, 256), 1)
    p_, q_ = (ri >> 7) & 1, (ri >> 6) & 1
    a_, b_ = (ri >> 3) & 7, ri & 7
    perm = ((2 * a_ + p_) * 16 + 2 * b_ + q_ == ci).astype(xb_ref.dtype)

    cat0[...] = jnp.zeros(cat0.shape, cat0.dtype)
    for n in range(B):
        y = jax.lax.dot_general(perm, xb_ref[n], (((1,), (1,)), ((), ())),
                                preferred_element_type=jnp.float32)
        y = y.astype(jnp.bfloat16)
        for a in range(8):
            cat0[n * 81 + 9 * (a + 1) + 1:n * 81 + 9 * (a + 1) + 9, :] = \
                jnp.concatenate([y[t * 64 + a * 8:t * 64 + a * 8 + 8, :]
                                 for t in range(4)], axis=1)

    acc0[...] = jnp.dot(cat0[0:M0, :], w0_ref[0],
                        preferred_element_type=jnp.float32)
    for t, off in ((1, 1), (2, 9), (3, 10)):
        acc0[...] += jnp.dot(cat0[off:off + M0, :], w0_ref[t],
                             preferred_element_type=jnp.float32)
    h0[...] = _lrelu(acc0[...] + b0_ref[...]).astype(jnp.bfloat16)

    sp1[...] = jnp.zeros(sp1.shape, sp1.dtype)
    hv = h0[...].reshape(B, 9, 9, Cp)[:, :8, :8, :].reshape(B, 4, 2, 4, 2, Cp)
    for p in range(2):
        for q in range(2):
            sp1[0:B, 1:5, 1:5, (2 * p + q) * Cp:(2 * p + q + 1) * Cp] = \
                hv[:, :, p, :, q, :]

    cp1.wait()
    v1 = sp1[...].reshape((B + 1) * 25, 4 * Cp)
    acc1[...] = jnp.dot(v1[0:M1], w1s[0], preferred_element_type=jnp.float32)
    for t, off in ((1, 1), (2, 5), (3, 6)):
        acc1[...] += jnp.dot(v1[off:off + M1], w1s[t],
                             preferred_element_type=jnp.float32)
    h1 = _lrelu(acc1[...] + b1_ref[...]).astype(jnp.bfloat16).reshape(B, 5, 5, Cp)

    hp[...] = jnp.zeros(hp.shape, hp.dtype)
    hp[0:B, 1:5, 1:5, :] = h1[:, 0:4, 0:4, :]
    cp2.wait()
    cp3.wait()
    cp4.wait()
    for i in (0, 1):
        for j in (0, 1):
            a = 2 * i + j
            col[a * B:(a + 1) * B, :] = \
                hp[0:B, 2 * i:2 * i + 3, 2 * j:2 * j + 3, :].reshape(B, 9 * Cp)
    hh = jnp.dot(col[...], tw1s[...],
                 preferred_element_type=jnp.float32) + tb1_ref[...]
    hh = _lrelu(hh).astype(jnp.bfloat16)
    hcat = jnp.swapaxes(hh.reshape(4, B, Cp), 0, 1).reshape(B, 4 * Cp)
    acc2 = jnp.dot(hcat, tw2s[...].reshape(4 * Cp, Cp),
                   preferred_element_type=jnp.float32)
    h2 = _lrelu(acc2 + tb2_ref[...]).astype(jnp.bfloat16)
    o_ref[...] = jnp.dot(h2, twls[...],
                         preferred_element_type=jnp.float32) + tbl_ref[...]


def kernel(x, conv0_w4, conv0_b4, conv1_w4, conv1_b4,
           tail_w1, tail_b1, tail_w2, tail_b2, tail_wl, tail_bl):
    N, Cin, H, W = x.shape
    assert H == 16 and W == 16 and N % 4 == 0, (N, Cin, H, W)
    K0 = conv0_w4.shape[1]
    Cp = conv0_w4.shape[2]
    assert K0 == 4 * Cin and conv1_w4.shape == (4, 4 * Cp, Cp)
    assert tail_w1.shape == (9 * Cp, Cp) and tail_wl.shape == (Cp, Cp)
    G = 4
    B = N // G
    M0 = B * 81

    xb = x.reshape(N, Cin, H * W)

    vspec = pl.BlockSpec(memory_space=pltpu.MemorySpace.VMEM)
    aspec = pl.BlockSpec(memory_space=pltpu.MemorySpace.HBM)
    out = pl.pallas_call(
        _fused_body,
        out_shape=jax.ShapeDtypeStruct((N, Cp), jnp.float32),
        grid=(G,),
        in_specs=[
            pl.BlockSpec((B, Cin, H * W), lambda i: (i, 0, 0)),
            vspec, vspec, aspec, vspec,
            aspec, vspec, aspec, vspec, aspec, vspec,
        ],
        out_specs=pl.BlockSpec((B, Cp), lambda i: (i, 0)),
        scratch_shapes=[
            pltpu.VMEM((4, 4 * Cp, Cp), jnp.bfloat16),
            pltpu.VMEM((9 * Cp, Cp), jnp.bfloat16),
            pltpu.VMEM((4, Cp, Cp), jnp.bfloat16),
            pltpu.VMEM((Cp, Cp), jnp.bfloat16),
            pltpu.SemaphoreType.DMA((4,)),
            pltpu.VMEM((M0 + 16, K0), jnp.bfloat16),
            pltpu.VMEM((M0, Cp), jnp.float32),
            pltpu.VMEM((M0, Cp), jnp.bfloat16),
            pltpu.VMEM((B + 1, 5, 5, 4 * Cp), jnp.bfloat16),
            pltpu.VMEM((B * 25, Cp), jnp.float32),
            pltpu.VMEM((B, 6, 6, Cp), jnp.bfloat16),
            pltpu.VMEM((4 * B, 9 * Cp), jnp.bfloat16),
        ],
        compiler_params=pltpu.CompilerParams(
            dimension_semantics=("parallel",),
            vmem_limit_bytes=int(62 << 20)),
        cost_estimate=pl.CostEstimate(
            flops=int(8 * (M0 * K0 + B * 25 * 4 * Cp + B * 9 * Cp) * Cp * 2 * G
                      // G + 2 * N * H * W * H * W * Cin),
            transcendentals=0,
            bytes_accessed=int(N * Cin * H * W * 4 + 2 * 24 * Cp * Cp)),
    )(xb, conv0_w4, conv0_b4, conv1_w4, conv1_b4,
      tail_w1, tail_b1, tail_w2, tail_b2, tail_wl, tail_bl)
    return out


# speedup vs baseline: 2.7181x; 1.0033x over previous
import jax
import jax.numpy as jnp
from jax.experimental import pallas as pl
from jax.experimental.pallas import tpu as pltpu

_SLOPE = 0.01


def _lrelu(v):
    return jnp.where(v >= 0, v, _SLOPE * v)


def _fused_body(xb_ref, w0_ref, b0_ref, w1_hbm, b1_ref,
                tw1_hbm, tb1_ref, tw2_hbm, tb2_ref, twl_hbm, tbl_ref,
                o_ref,
                w1s, tw1s, tw2s, twls, sems,
                cat0, acc0, h0, sp1, acc1, hp, col):
    B = o_ref.shape[0]
    Cin = xb_ref.shape[1]
    K0 = 4 * Cin
    Cp = w0_ref.shape[2]
    M0 = B * 81
    M1 = B * 25

    first = pl.program_id(0) == 0
    cp1 = pltpu.make_async_copy(w1_hbm, w1s, sems.at[0])
    cp2 = pltpu.make_async_copy(tw1_hbm, tw1s, sems.at[1])
    cp3 = pltpu.make_async_copy(tw2_hbm, tw2s, sems.at[2])
    cp4 = pltpu.make_async_copy(twl_hbm, twls, sems.at[3])

    @pl.when(first)
    def _():
        cp1.start()
        cp2.start()
        cp3.start()
        cp4.start()

    ri = jax.lax.broadcasted_iota(jnp.int32, (256, 256), 0)
    ci = jax.lax.broadcasted_iota(jnp.int32, (256, 256), 1)
    p_, q_ = (ri >> 7) & 1, (ri >> 6) & 1
    a_, b_ = (ri >> 3) & 7, ri & 7
    perm = ((2 * a_ + p_) * 16 + 2 * b_ + q_ == ci).astype(xb_ref.dtype)

    cat0[...] = jnp.zeros(cat0.shape, cat0.dtype)
    for n in range(B):
        y = jax.lax.dot_general(perm, xb_ref[n], (((1,), (1,)), ((), ())),
                                preferred_element_type=jnp.float32)
        y = y.astype(jnp.bfloat16)
        for a in range(8):
            cat0[n * 81 + 9 * (a + 1) + 1:n * 81 + 9 * (a + 1) + 9, :] = \
                jnp.concatenate([y[t * 64 + a * 8:t * 64 + a * 8 + 8, :]
                                 for t in range(4)], axis=1)

    acc0[...] = jnp.dot(cat0[0:M0, :], w0_ref[0],
                        preferred_element_type=jnp.float32)
    for t, off in ((1, 1), (2, 9), (3, 10)):
        acc0[...] += jnp.dot(cat0[off:off + M0, :], w0_ref[t],
                             preferred_element_type=jnp.float32)
    h0[...] = _lrelu(acc0[...] + b0_ref[...]).astype(jnp.bfloat16)

    sp1[...] = jnp.zeros(sp1.shape, sp1.dtype)
    hv = h0[...].reshape(B, 9, 9, Cp)[:, :8, :8, :].reshape(B, 4, 2, 4, 2, Cp)
    for p in range(2):
        for q in range(2):
            sp1[0:B, 1:5, 1:5, (2 * p + q) * Cp:(2 * p + q + 1) * Cp] = \
                hv[:, :, p, :, q, :]

    @pl.when(first)
    def _():
        cp1.wait()
    v1 = sp1[...].reshape((B + 1) * 25, 4 * Cp)
    acc1[...] = jnp.dot(v1[0:M1], w1s[0], preferred_element_type=jnp.float32)
    for t, off in ((1, 1), (2, 5), (3, 6)):
        acc1[...] += jnp.dot(v1[off:off + M1], w1s[t],
                             preferred_element_type=jnp.float32)
    h1 = _lrelu(acc1[...] + b1_ref[...]).astype(jnp.bfloat16).reshape(B, 5, 5, Cp)

    hp[...] = jnp.zeros(hp.shape, hp.dtype)
    hp[0:B, 1:5, 1:5, :] = h1[:, 0:4, 0:4, :]
    @pl.when(first)
    def _():
        cp2.wait()
        cp3.wait()
        cp4.wait()
    for i in (0, 1):
        for j in (0, 1):
            a = 2 * i + j
            col[a * B:(a + 1) * B, :] = \
                hp[0:B, 2 * i:2 * i + 3, 2 * j:2 * j + 3, :].reshape(B, 9 * Cp)
    hh = jnp.dot(col[...], tw1s[...],
                 preferred_element_type=jnp.float32) + tb1_ref[...]
    hh = _lrelu(hh).astype(jnp.bfloat16)
    hcat = jnp.swapaxes(hh.reshape(4, B, Cp), 0, 1).reshape(B, 4 * Cp)
    acc2 = jnp.dot(hcat, tw2s[...].reshape(4 * Cp, Cp),
                   preferred_element_type=jnp.float32)
    h2 = _lrelu(acc2 + tb2_ref[...]).astype(jnp.bfloat16)
    o_ref[...] = jnp.dot(h2, twls[...],
                         preferred_element_type=jnp.float32) + tbl_ref[...]


def kernel(x, conv0_w4, conv0_b4, conv1_w4, conv1_b4,
           tail_w1, tail_b1, tail_w2, tail_b2, tail_wl, tail_bl):
    N, Cin, H, W = x.shape
    assert H == 16 and W == 16 and N % 4 == 0, (N, Cin, H, W)
    K0 = conv0_w4.shape[1]
    Cp = conv0_w4.shape[2]
    assert K0 == 4 * Cin and conv1_w4.shape == (4, 4 * Cp, Cp)
    assert tail_w1.shape == (9 * Cp, Cp) and tail_wl.shape == (Cp, Cp)
    G = 4
    B = N // G
    M0 = B * 81

    xb = x.reshape(N, Cin, H * W)

    vspec = pl.BlockSpec(memory_space=pltpu.MemorySpace.VMEM)
    aspec = pl.BlockSpec(memory_space=pltpu.MemorySpace.HBM)
    out = pl.pallas_call(
        _fused_body,
        out_shape=jax.ShapeDtypeStruct((N, Cp), jnp.float32),
        grid=(G,),
        in_specs=[
            pl.BlockSpec((B, Cin, H * W), lambda i: (i, 0, 0)),
            vspec, vspec, aspec, vspec,
            aspec, vspec, aspec, vspec, aspec, vspec,
        ],
        out_specs=pl.BlockSpec((B, Cp), lambda i: (i, 0)),
        scratch_shapes=[
            pltpu.VMEM((4, 4 * Cp, Cp), jnp.bfloat16),
            pltpu.VMEM((9 * Cp, Cp), jnp.bfloat16),
            pltpu.VMEM((4, Cp, Cp), jnp.bfloat16),
            pltpu.VMEM((Cp, Cp), jnp.bfloat16),
            pltpu.SemaphoreType.DMA((4,)),
            pltpu.VMEM((M0 + 16, K0), jnp.bfloat16),
            pltpu.VMEM((M0, Cp), jnp.float32),
            pltpu.VMEM((M0, Cp), jnp.bfloat16),
            pltpu.VMEM((B + 1, 5, 5, 4 * Cp), jnp.bfloat16),
            pltpu.VMEM((B * 25, Cp), jnp.float32),
            pltpu.VMEM((B, 6, 6, Cp), jnp.bfloat16),
            pltpu.VMEM((4 * B, 9 * Cp), jnp.bfloat16),
        ],
        compiler_params=pltpu.CompilerParams(
            dimension_semantics=("arbitrary",),
            vmem_limit_bytes=int(62 << 20)),
        cost_estimate=pl.CostEstimate(
            flops=int(8 * (M0 * K0 + B * 25 * 4 * Cp + B * 9 * Cp) * Cp * 2 * G
                      // G + 2 * N * H * W * H * W * Cin),
            transcendentals=0,
            bytes_accessed=int(N * Cin * H * W * 4 + 2 * 24 * Cp * Cp)),
    )(xb, conv0_w4, conv0_b4, conv1_w4, conv1_b4,
      tail_w1, tail_b1, tail_w2, tail_b2, tail_wl, tail_bl)
    return out


# speedup vs baseline: 2.8417x; 1.0455x over previous
import jax
import jax.numpy as jnp
from jax.experimental import pallas as pl
from jax.experimental.pallas import tpu as pltpu

_SLOPE = 0.01


def _lrelu(v):
    return jnp.where(v >= 0, v, _SLOPE * v)


def _fused_body(xb_ref, w0_ref, b0_ref, w1_hbm, b1_ref,
                tw1_hbm, tb1_ref, tw2_hbm, tb2_ref, twl_hbm, tbl_ref,
                o_ref,
                w1s, tw1s, tw2s, twls, sems,
                cat0, acc0, h0, sp1, acc1, hp, col):
    B = o_ref.shape[0]
    Cin = xb_ref.shape[1]
    K0 = 4 * Cin
    Cp = w0_ref.shape[2]
    M0 = B * 81
    M1 = B * 25

    first = pl.program_id(0) == 0
    cp1 = pltpu.make_async_copy(w1_hbm, w1s, sems.at[0])
    cp2 = pltpu.make_async_copy(tw1_hbm, tw1s, sems.at[1])
    cp3 = pltpu.make_async_copy(tw2_hbm, tw2s, sems.at[2])
    cp4 = pltpu.make_async_copy(twl_hbm, twls, sems.at[3])

    @pl.when(first)
    def _():
        cp1.start()
        cp2.start()
        cp3.start()
        cp4.start()

    ri = jax.lax.broadcasted_iota(jnp.int32, (256, 256), 0)
    ci = jax.lax.broadcasted_iota(jnp.int32, (256, 256), 1)
    p_, q_ = (ri >> 7) & 1, (ri >> 6) & 1
    a_, b_ = (ri >> 3) & 7, ri & 7
    perm = ((2 * a_ + p_) * 16 + 2 * b_ + q_ == ci).astype(xb_ref.dtype)

    cat0[...] = jnp.zeros(cat0.shape, cat0.dtype)
    for n in range(B):
        y = jax.lax.dot_general(perm, xb_ref[n], (((1,), (1,)), ((), ())),
                                preferred_element_type=jnp.float32)
        y = y.astype(jnp.bfloat16)
        for a in range(8):
            cat0[n * 81 + 9 * (a + 1) + 1:n * 81 + 9 * (a + 1) + 9, :] = \
                jnp.concatenate([y[t * 64 + a * 8:t * 64 + a * 8 + 8, :]
                                 for t in range(4)], axis=1)

    acc0[...] = jnp.dot(cat0[0:M0, :], w0_ref[0],
                        preferred_element_type=jnp.float32)
    for t, off in ((1, 1), (2, 9), (3, 10)):
        acc0[...] += jnp.dot(cat0[off:off + M0, :], w0_ref[t],
                             preferred_element_type=jnp.float32)
    h0[...] = _lrelu(acc0[...] + b0_ref[...]).astype(jnp.bfloat16)

    sp1[...] = jnp.zeros(sp1.shape, sp1.dtype)
    hv = h0[...].reshape(B, 9, 9, Cp)[:, :8, :8, :].reshape(B, 4, 2, 4, 2, Cp)
    for p in range(2):
        for q in range(2):
            sp1[0:B, 1:5, 1:5, (2 * p + q) * Cp:(2 * p + q + 1) * Cp] = \
                hv[:, :, p, :, q, :]

    @pl.when(first)
    def _():
        cp1.wait()
    v1 = sp1[...].reshape((B + 1) * 25, 4 * Cp)
    acc1[...] = jnp.dot(v1[0:M1], w1s[0], preferred_element_type=jnp.float32)
    for t, off in ((1, 1), (2, 5), (3, 6)):
        acc1[...] += jnp.dot(v1[off:off + M1], w1s[t],
                             preferred_element_type=jnp.float32)
    h1 = _lrelu(acc1[...] + b1_ref[...]).astype(jnp.bfloat16).reshape(B, 5, 5, Cp)

    hp[...] = jnp.zeros(hp.shape, hp.dtype)
    hp[0:B, 1:5, 1:5, :] = h1[:, 0:4, 0:4, :]
    @pl.when(first)
    def _():
        cp2.wait()
        cp3.wait()
        cp4.wait()
    for i in (0, 1):
        for j in (0, 1):
            a = 2 * i + j
            col[a * B:(a + 1) * B, :] = \
                hp[0:B, 2 * i:2 * i + 3, 2 * j:2 * j + 3, :].reshape(B, 9 * Cp)
    hh = jnp.dot(col[...], tw1s[...],
                 preferred_element_type=jnp.float32) + tb1_ref[...]
    hh = _lrelu(hh).astype(jnp.bfloat16)
    hcat = jnp.swapaxes(hh.reshape(4, B, Cp), 0, 1).reshape(B, 4 * Cp)
    acc2 = jnp.dot(hcat, tw2s[...].reshape(4 * Cp, Cp),
                   preferred_element_type=jnp.float32)
    h2 = _lrelu(acc2 + tb2_ref[...]).astype(jnp.bfloat16)
    o_ref[...] = jnp.dot(h2, twls[...],
                         preferred_element_type=jnp.float32) + tbl_ref[...]


def kernel(x, conv0_w4, conv0_b4, conv1_w4, conv1_b4,
           tail_w1, tail_b1, tail_w2, tail_b2, tail_wl, tail_bl):
    N, Cin, H, W = x.shape
    assert H == 16 and W == 16 and N % 2 == 0, (N, Cin, H, W)
    K0 = conv0_w4.shape[1]
    Cp = conv0_w4.shape[2]
    assert K0 == 4 * Cin and conv1_w4.shape == (4, 4 * Cp, Cp)
    assert tail_w1.shape == (9 * Cp, Cp) and tail_wl.shape == (Cp, Cp)
    G = 2
    B = N // G
    M0 = B * 81

    xb = x.reshape(N, Cin, H * W)

    vspec = pl.BlockSpec(memory_space=pltpu.MemorySpace.VMEM)
    aspec = pl.BlockSpec(memory_space=pltpu.MemorySpace.HBM)
    out = pl.pallas_call(
        _fused_body,
        out_shape=jax.ShapeDtypeStruct((N, Cp), jnp.float32),
        grid=(G,),
        in_specs=[
            pl.BlockSpec((B, Cin, H * W), lambda i: (i, 0, 0)),
            vspec, vspec, aspec, vspec,
            aspec, vspec, aspec, vspec, aspec, vspec,
        ],
        out_specs=pl.BlockSpec((B, Cp), lambda i: (i, 0)),
        scratch_shapes=[
            pltpu.VMEM((4, 4 * Cp, Cp), jnp.bfloat16),
            pltpu.VMEM((9 * Cp, Cp), jnp.bfloat16),
            pltpu.VMEM((4, Cp, Cp), jnp.bfloat16),
            pltpu.VMEM((Cp, Cp), jnp.bfloat16),
            pltpu.SemaphoreType.DMA((4,)),
            pltpu.VMEM((M0 + 16, K0), jnp.bfloat16),
            pltpu.VMEM((M0, Cp), jnp.float32),
            pltpu.VMEM((M0, Cp), jnp.bfloat16),
            pltpu.VMEM((B + 1, 5, 5, 4 * Cp), jnp.bfloat16),
            pltpu.VMEM((B * 25, Cp), jnp.float32),
            pltpu.VMEM((B, 6, 6, Cp), jnp.bfloat16),
            pltpu.VMEM((4 * B, 9 * Cp), jnp.bfloat16),
        ],
        compiler_params=pltpu.CompilerParams(
            dimension_semantics=("arbitrary",),
            vmem_limit_bytes=int(62 << 20)),
        cost_estimate=pl.CostEstimate(
            flops=int(8 * (M0 * K0 + B * 25 * 4 * Cp + B * 9 * Cp) * Cp * 2 * G
                      // G + 2 * N * H * W * H * W * Cin),
            transcendentals=0,
            bytes_accessed=int(N * Cin * H * W * 4 + 2 * 24 * Cp * Cp)),
    )(xb, conv0_w4, conv0_b4, conv1_w4, conv1_b4,
      tail_w1, tail_b1, tail_w2, tail_b2, tail_wl, tail_bl)
    return out
